# baseline (speedup 1.0000x reference)
import functools

import jax
import jax.numpy as jnp
from jax import lax
from jax.experimental import pallas as pl
from jax.experimental.pallas import tpu as pltpu
from jax.experimental.pallas import tpu_sc as plsc

N = 262144
D = 64
LANES = 16

NS = 16
CHUNK = N // NS
NHALF = 2
HALF = CHUNK // NHALF
HVREG = HALF // LANES
BITS = 10
BINS = 1 << BITS
SCAT = 128
HROWS = HALF // SCAT
KGRP = 8

NW = 32
RPW = N // NW
GC = 64
NG = RPW // GC

_mesh = plsc.VectorSubcoreMesh(core_axis_name="c", subcore_axis_name="s")


@functools.partial(
    pl.kernel,
    out_type=(
        jax.ShapeDtypeStruct((N,), jnp.int32),
        jax.ShapeDtypeStruct((N, D), jnp.float32),
    ),
    mesh=_mesh,
    compiler_params=pltpu.CompilerParams(
        needs_layout_passes=False, use_tc_tiling_on_sc=False),
    scratch_types=[
        pltpu.VMEM((HALF,), jnp.int32),
        pltpu.VMEM((HALF,), jnp.int32),
        pltpu.VMEM((BINS,), jnp.int32),
        pltpu.VMEM((NS, BINS), jnp.int32),
        pltpu.VMEM((BINS,), jnp.int32),
        pltpu.VMEM((HROWS, SCAT), jnp.int32),
        pltpu.VMEM((RPW,), jnp.int32),
        pltpu.VMEM((GC, D), jnp.float32),
        pltpu.VMEM((GC, D), jnp.float32),
        pltpu.SemaphoreType.DMA,
        pltpu.SemaphoreType.DMA,
        pltpu.VMEM_SHARED((N,), jnp.int32),
        pltpu.VMEM_SHARED((N,), jnp.int32),
        pltpu.VMEM_SHARED((N,), jnp.int32),
        pltpu.VMEM_SHARED((N,), jnp.int32),
        pltpu.VMEM_SHARED((NS, BINS), jnp.int32),
    ],
)
def _sorter_kernel(phi_hbm, emb_hbm, out_phi, out_emb,
                   loc_k, loc_i, hist_v, grid_v, offs_v, pos_v,
                   idx_v, rows_a, rows_b, sem_k, sem_i,
                   sp_k0, sp_i0, sp_k1, sp_i1, sp_hist):
    t = lax.axis_index("s")
    c = lax.axis_index("c")
    base = t * CHUNK
    zeros16 = lax.full((LANES,), 0, jnp.int32)

    def digit_of(u, shift):
        return lax.shift_right_logical(
            u, lax.full((LANES,), shift, jnp.int32)) & (BINS - 1)

    def run_pass(shift, src_k, src_i, dst_k, dst_i, first):
        def zeroh(j, carry):
            hist_v[pl.ds(j * LANES, LANES)] = zeros16
            return carry
        lax.fori_loop(0, BINS // LANES, zeroh, 0)

        def histg(g, carry):
            dcl = []
            for jj in range(SCAT // LANES):
                d = digit_of(loc_k[pl.ds(g * SCAT + jj * LANES, LANES)],
                             shift)
                cnt, last = plsc.scan_count(d)
                dcl.append((d, cnt, last))
            for d, cnt, last in dcl:
                plsc.addupdate_scatter(hist_v, [d], cnt, mask=last)
            return carry

        for h in range(NHALF):
            pltpu.sync_copy(src_k.at[pl.ds(base + h * HALF, HALF)], loc_k)
            lax.fori_loop(0, HROWS, histg, 0)

        pltpu.sync_copy(hist_v, sp_hist.at[t])
        plsc.subcore_barrier()
        pltpu.sync_copy(sp_hist, grid_v)

        def offj(j, carry):
            tot = zeros16
            rp = zeros16
            for tt in range(NS):
                row = grid_v[tt, pl.ds(j * LANES, LANES)]
                tot = tot + row
                w = jnp.where(tt < t, jnp.int32(1), jnp.int32(0))
                rp = rp + row * w
            excl = plsc.cumsum(tot) - tot
            offs_v[pl.ds(j * LANES, LANES)] = excl + rp + carry
            return carry + jnp.sum(tot)
        lax.fori_loop(0, BINS // LANES, offj, jnp.int32(0))

        def drain_group(g0):
            for rr in range(KGRP):
                r = g0 * KGRP + rr
                pltpu.make_async_copy(loc_k.at[pl.ds(r * SCAT, SCAT)],
                                      dst_k.at[pos_v.at[r]], sem_k).wait()
                pltpu.make_async_copy(loc_i.at[pl.ds(r * SCAT, SCAT)],
                                      dst_i.at[pos_v.at[r]], sem_i).wait()

        def permg(g, carry):
            @pl.when(g > 0)
            def _():
                drain_group(g - 1)
            for rr in range(KGRP):
                r = g * KGRP + rr
                dcl = []
                for jj in range(SCAT // LANES):
                    d = digit_of(
                        loc_k[pl.ds(r * SCAT + jj * LANES, LANES)], shift)
                    cnt, last = plsc.scan_count(d)
                    dcl.append((d, cnt, last))
                for jj, (d, cnt, last) in enumerate(dcl):
                    pos = plsc.load_gather(offs_v, [d]) + cnt - 1
                    plsc.addupdate_scatter(offs_v, [d], cnt, mask=last)
                    pos_v[r, pl.ds(jj * LANES, LANES)] = pos
            for rr in range(KGRP):
                r = g * KGRP + rr
                pltpu.async_copy(loc_k.at[pl.ds(r * SCAT, SCAT)],
                                 dst_k.at[pos_v.at[r]], sem_k)
                pltpu.async_copy(loc_i.at[pl.ds(r * SCAT, SCAT)],
                                 dst_i.at[pos_v.at[r]], sem_i)
            return carry

        for h in range(NHALF):
            hbase = base + h * HALF
            pltpu.sync_copy(src_k.at[pl.ds(hbase, HALF)], loc_k)
            if first:
                def fill(j, carry):
                    loc_i[pl.ds(j * LANES, LANES)] = (
                        hbase + j * LANES + lax.iota(jnp.int32, LANES))
                    return carry
                lax.fori_loop(0, HVREG, fill, 0)
            else:
                pltpu.sync_copy(src_i.at[pl.ds(hbase, HALF)], loc_i)
            lax.fori_loop(0, HROWS // KGRP, permg, 0)
            drain_group(HROWS // KGRP - 1)
        plsc.subcore_barrier()

    run_pass(0, phi_hbm, None, sp_k0, sp_i0, True)
    run_pass(BITS, sp_k0, sp_i0, sp_k1, sp_i1, False)
    run_pass(2 * BITS, sp_k1, sp_i1, sp_k0, sp_i0, False)

    wid = c * NS + t
    gbase = wid * RPW
    pltpu.sync_copy(sp_k0.at[pl.ds(gbase, RPW)],
                    out_phi.at[pl.ds(gbase, RPW)])
    pltpu.sync_copy(sp_i0.at[pl.ds(gbase, RPW)], idx_v)

    pltpu.async_copy(emb_hbm.at[idx_v.at[pl.ds(0, GC)]], rows_a, sem_k)

    def g(gi, carry):
        r = 2 * gi
        pltpu.async_copy(emb_hbm.at[idx_v.at[pl.ds((r + 1) * GC, GC)]],
                         rows_b, sem_i)
        pltpu.make_async_copy(emb_hbm.at[idx_v.at[pl.ds(r * GC, GC)]],
                              rows_a, sem_k).wait()
        pltpu.sync_copy(rows_a, out_emb.at[pl.ds(gbase + r * GC, GC)])

        @pl.when(gi + 1 < NG // 2)
        def _():
            pltpu.async_copy(emb_hbm.at[idx_v.at[pl.ds((r + 2) * GC, GC)]],
                             rows_a, sem_k)
        pltpu.make_async_copy(emb_hbm.at[idx_v.at[pl.ds((r + 1) * GC, GC)]],
                              rows_b, sem_i).wait()
        pltpu.sync_copy(rows_b, out_emb.at[pl.ds(gbase + (r + 1) * GC, GC)])
        return carry
    lax.fori_loop(0, NG // 2, g, 0)


def kernel(key_phi, key_embed):
    phi_bits = lax.bitcast_convert_type(key_phi.reshape(N), jnp.int32)
    emb = key_embed.reshape(N, D)
    sorted_bits, sorted_emb = _sorter_kernel(phi_bits, emb)
    sorted_phi = lax.bitcast_convert_type(sorted_bits, jnp.float32)
    return sorted_phi.reshape(1, N), sorted_emb.reshape(1, N, D)


# speedup vs baseline: 1.2857x; 1.2857x over previous
import functools

import jax
import jax.numpy as jnp
from jax import lax
from jax.experimental import pallas as pl
from jax.experimental.pallas import tpu as pltpu
from jax.experimental.pallas import tpu_sc as plsc

N = 262144
D = 64
LANES = 16

NS = 16
CHUNK = N // NS
NHALF = 2
HALF = CHUNK // NHALF
HVREG = HALF // LANES
BITS = 10
BINS = 1 << BITS
SCAT = 128
HROWS = HALF // SCAT
KGRP = 8

NW = 32
RPW = N // NW
GC = 128
NG = RPW // GC

_sort_mesh = plsc.VectorSubcoreMesh(
    core_axis_name="c", subcore_axis_name="s", num_cores=1)
_gather_mesh = plsc.VectorSubcoreMesh(
    core_axis_name="c", subcore_axis_name="s")


@functools.partial(
    pl.kernel,
    out_type=(
        jax.ShapeDtypeStruct((N,), jnp.int32),
        jax.ShapeDtypeStruct((N,), jnp.int32),
    ),
    mesh=_sort_mesh,
    compiler_params=pltpu.CompilerParams(needs_layout_passes=False),
    scratch_types=[
        pltpu.VMEM((HALF,), jnp.int32),
        pltpu.VMEM((HALF,), jnp.int32),
        pltpu.VMEM((BINS,), jnp.int32),
        pltpu.VMEM((NS, BINS), jnp.int32),
        pltpu.VMEM((BINS,), jnp.int32),
        pltpu.VMEM((HROWS, SCAT), jnp.int32),
        pltpu.SemaphoreType.DMA,
        pltpu.SemaphoreType.DMA,
        pltpu.VMEM_SHARED((N,), jnp.int32),
        pltpu.VMEM_SHARED((N,), jnp.int32),
        pltpu.VMEM_SHARED((N,), jnp.int32),
        pltpu.VMEM_SHARED((N,), jnp.int32),
        pltpu.VMEM_SHARED((NS, BINS), jnp.int32),
    ],
)
def _sort_kernel(phi_hbm, out_phi, out_idx,
                 loc_k, loc_i, hist_v, grid_v, offs_v, pos_v, sem_k, sem_i,
                 sp_k0, sp_i0, sp_k1, sp_i1, sp_hist):
    t = lax.axis_index("s")
    base = t * CHUNK
    zeros16 = lax.full((LANES,), 0, jnp.int32)

    def digit_of(u, shift):
        return lax.shift_right_logical(u, lax.full((LANES,), shift, jnp.int32)) \
            & (BINS - 1)

    def run_pass(shift, src_k, src_i, dst_k, dst_i, first):
        def zeroh(j, carry):
            hist_v[pl.ds(j * LANES, LANES)] = zeros16
            return carry
        lax.fori_loop(0, BINS // LANES, zeroh, 0)

        def histg(g, carry):
            dcl = []
            for jj in range(SCAT // LANES):
                d = digit_of(loc_k[pl.ds(g * SCAT + jj * LANES, LANES)],
                             shift)
                cnt, last = plsc.scan_count(d)
                dcl.append((d, cnt, last))
            for d, cnt, last in dcl:
                plsc.addupdate_scatter(hist_v, [d], cnt, mask=last)
            return carry

        for h in range(NHALF):
            pltpu.sync_copy(src_k.at[pl.ds(base + h * HALF, HALF)], loc_k)
            lax.fori_loop(0, HROWS, histg, 0)

        pltpu.sync_copy(hist_v, sp_hist.at[t])
        plsc.subcore_barrier()
        pltpu.sync_copy(sp_hist, grid_v)

        def offj(j, carry):
            tot = zeros16
            rp = zeros16
            for tt in range(NS):
                row = grid_v[tt, pl.ds(j * LANES, LANES)]
                tot = tot + row
                w = jnp.where(tt < t, jnp.int32(1), jnp.int32(0))
                rp = rp + row * w
            excl = plsc.cumsum(tot) - tot
            offs_v[pl.ds(j * LANES, LANES)] = excl + rp + carry
            return carry + jnp.sum(tot)
        lax.fori_loop(0, BINS // LANES, offj, jnp.int32(0))

        def drain_group(g0):
            for rr in range(KGRP):
                r = g0 * KGRP + rr
                pltpu.make_async_copy(loc_k.at[pl.ds(r * SCAT, SCAT)],
                                      dst_k.at[pos_v.at[r]], sem_k).wait()
                pltpu.make_async_copy(loc_i.at[pl.ds(r * SCAT, SCAT)],
                                      dst_i.at[pos_v.at[r]], sem_i).wait()

        def permg(g, carry):
            @pl.when(g > 0)
            def _():
                drain_group(g - 1)
            for rr in range(KGRP):
                r = g * KGRP + rr
                dcl = []
                for jj in range(SCAT // LANES):
                    d = digit_of(
                        loc_k[pl.ds(r * SCAT + jj * LANES, LANES)], shift)
                    cnt, last = plsc.scan_count(d)
                    dcl.append((d, cnt, last))
                for jj, (d, cnt, last) in enumerate(dcl):
                    pos = plsc.load_gather(offs_v, [d]) + cnt - 1
                    plsc.addupdate_scatter(offs_v, [d], cnt, mask=last)
                    pos_v[r, pl.ds(jj * LANES, LANES)] = pos
            for rr in range(KGRP):
                r = g * KGRP + rr
                pltpu.async_copy(loc_k.at[pl.ds(r * SCAT, SCAT)],
                                 dst_k.at[pos_v.at[r]], sem_k)
                pltpu.async_copy(loc_i.at[pl.ds(r * SCAT, SCAT)],
                                 dst_i.at[pos_v.at[r]], sem_i)
            return carry

        for h in range(NHALF):
            hbase = base + h * HALF
            pltpu.sync_copy(src_k.at[pl.ds(hbase, HALF)], loc_k)
            if first:
                def fill(j, carry):
                    loc_i[pl.ds(j * LANES, LANES)] = (
                        hbase + j * LANES + lax.iota(jnp.int32, LANES))
                    return carry
                lax.fori_loop(0, HVREG, fill, 0)
            else:
                pltpu.sync_copy(src_i.at[pl.ds(hbase, HALF)], loc_i)
            lax.fori_loop(0, HROWS // KGRP, permg, 0)
            drain_group(HROWS // KGRP - 1)
        plsc.subcore_barrier()

    run_pass(0, phi_hbm, None, sp_k0, sp_i0, True)
    run_pass(BITS, sp_k0, sp_i0, sp_k1, sp_i1, False)
    run_pass(2 * BITS, sp_k1, sp_i1, sp_k0, sp_i0, False)

    pltpu.sync_copy(sp_k0.at[pl.ds(base, CHUNK)],
                    out_phi.at[pl.ds(base, CHUNK)])
    pltpu.sync_copy(sp_i0.at[pl.ds(base, CHUNK)],
                    out_idx.at[pl.ds(base, CHUNK)])


@functools.partial(
    pl.kernel,
    out_type=jax.ShapeDtypeStruct((N, D), jnp.float32),
    mesh=_gather_mesh,
    compiler_params=pltpu.CompilerParams(
        needs_layout_passes=False, use_tc_tiling_on_sc=False),
    scratch_types=[
        pltpu.VMEM((NG, GC), jnp.int32),
        pltpu.VMEM((GC, D), jnp.float32),
        pltpu.VMEM((GC, D), jnp.float32),
        pltpu.SemaphoreType.DMA,
        pltpu.SemaphoreType.DMA,
    ],
)
def _gather_kernel(emb_hbm, idx_hbm, out_hbm, idx_v, rows_a, rows_b,
                   sem_a, sem_b):
    wid = lax.axis_index("s") * 2 + lax.axis_index("c")
    pltpu.sync_copy(idx_hbm.at[pl.ds(wid * NG, NG)], idx_v)
    base = wid * RPW

    pltpu.async_copy(emb_hbm.at[idx_v.at[0]], rows_a, sem_a)

    def g(gi, carry):
        r = 2 * gi
        pltpu.async_copy(emb_hbm.at[idx_v.at[r + 1]], rows_b, sem_b)
        pltpu.make_async_copy(emb_hbm.at[idx_v.at[r]], rows_a, sem_a).wait()
        pltpu.sync_copy(rows_a, out_hbm.at[pl.ds(base + r * GC, GC)])

        @pl.when(gi + 1 < NG // 2)
        def _():
            pltpu.async_copy(emb_hbm.at[idx_v.at[r + 2]], rows_a, sem_a)
        pltpu.make_async_copy(emb_hbm.at[idx_v.at[r + 1]], rows_b,
                              sem_b).wait()
        pltpu.sync_copy(rows_b, out_hbm.at[pl.ds(base + (r + 1) * GC, GC)])
        return carry
    lax.fori_loop(0, NG // 2, g, 0)


def kernel(key_phi, key_embed):
    phi_bits = lax.bitcast_convert_type(key_phi.reshape(N), jnp.int32)
    emb_flat = lax.optimization_barrier(key_embed.reshape(N * D))
    emb = emb_flat.reshape(N, D)
    sorted_bits, sort_idx = _sort_kernel(phi_bits)
    sorted_phi = lax.bitcast_convert_type(sorted_bits, jnp.float32)
    sorted_emb = _gather_kernel(emb, sort_idx.reshape(N // GC, GC))
    out_flat = lax.optimization_barrier(sorted_emb.reshape(N * D))
    return sorted_phi.reshape(1, N), out_flat.reshape(1, N, D)


# speedup vs baseline: 1.3191x; 1.0260x over previous
import functools

import jax
import jax.numpy as jnp
from jax import lax
from jax.experimental import pallas as pl
from jax.experimental.pallas import tpu as pltpu
from jax.experimental.pallas import tpu_sc as plsc

N = 262144
D = 64
LANES = 16

NS = 16
CHUNK = N // NS
NHALF = 2
HALF = CHUNK // NHALF
HVREG = HALF // LANES
BITS = 10
BINS = 1 << BITS
SCAT = 128
HROWS = HALF // SCAT
KGRP = 8

NW = 32
RPW = N // NW
GC = 128
NG = RPW // GC

_sort_mesh = plsc.VectorSubcoreMesh(
    core_axis_name="c", subcore_axis_name="s", num_cores=1)
_gather_mesh = plsc.VectorSubcoreMesh(
    core_axis_name="c", subcore_axis_name="s")


@functools.partial(
    pl.kernel,
    out_type=(
        jax.ShapeDtypeStruct((N,), jnp.int32),
        jax.ShapeDtypeStruct((N,), jnp.int32),
    ),
    mesh=_sort_mesh,
    compiler_params=pltpu.CompilerParams(needs_layout_passes=False),
    scratch_types=[
        pltpu.VMEM((HALF,), jnp.int32),
        pltpu.VMEM((HALF,), jnp.int32),
        pltpu.VMEM((BINS,), jnp.int32),
        pltpu.VMEM((NS, BINS), jnp.int32),
        pltpu.VMEM((BINS,), jnp.int32),
        pltpu.VMEM((HROWS, SCAT), jnp.int32),
        pltpu.SemaphoreType.DMA,
        pltpu.SemaphoreType.DMA,
        pltpu.VMEM_SHARED((N,), jnp.int32),
        pltpu.VMEM_SHARED((N,), jnp.int32),
        pltpu.VMEM_SHARED((N,), jnp.int32),
        pltpu.VMEM_SHARED((N,), jnp.int32),
        pltpu.VMEM_SHARED((NS, BINS), jnp.int32),
    ],
)
def _sort_kernel(phi_hbm, out_phi, out_idx,
                 loc_k, loc_i, hist_v, grid_v, offs_v, pos_v, sem_k, sem_i,
                 sp_k0, sp_i0, sp_k1, sp_i1, sp_hist):
    t = lax.axis_index("s")
    base = t * CHUNK
    zeros16 = lax.full((LANES,), 0, jnp.int32)

    def digit_of(u, shift):
        return lax.shift_right_logical(u, lax.full((LANES,), shift, jnp.int32)) \
            & (BINS - 1)

    def run_pass(shift, src_k, src_i, dst_k, dst_i, first):
        def zeroh(j, carry):
            hist_v[pl.ds(j * LANES, LANES)] = zeros16
            return carry
        lax.fori_loop(0, BINS // LANES, zeroh, 0)

        def histg(g, carry):
            dcl = []
            for jj in range(SCAT // LANES):
                d = digit_of(loc_k[pl.ds(g * SCAT + jj * LANES, LANES)],
                             shift)
                cnt, last = plsc.scan_count(d)
                dcl.append((d, cnt, last))
            for d, cnt, last in dcl:
                plsc.addupdate_scatter(hist_v, [d], cnt, mask=last)
            return carry

        for h in range(NHALF):
            pltpu.sync_copy(src_k.at[pl.ds(base + h * HALF, HALF)], loc_k)
            lax.fori_loop(0, HROWS, histg, 0)

        pltpu.sync_copy(hist_v, sp_hist.at[t])
        plsc.subcore_barrier()
        pltpu.sync_copy(sp_hist, grid_v)

        def offj(j, carry):
            tot = zeros16
            rp = zeros16
            for tt in range(NS):
                row = grid_v[tt, pl.ds(j * LANES, LANES)]
                tot = tot + row
                w = jnp.where(tt < t, jnp.int32(1), jnp.int32(0))
                rp = rp + row * w
            excl = plsc.cumsum(tot) - tot
            offs_v[pl.ds(j * LANES, LANES)] = excl + rp + carry
            return carry + jnp.sum(tot)
        lax.fori_loop(0, BINS // LANES, offj, jnp.int32(0))

        def drain_group(g0):
            for rr in range(KGRP):
                r = g0 * KGRP + rr
                pltpu.make_async_copy(loc_k.at[pl.ds(r * SCAT, SCAT)],
                                      dst_k.at[pos_v.at[r]], sem_k).wait()
                pltpu.make_async_copy(loc_i.at[pl.ds(r * SCAT, SCAT)],
                                      dst_i.at[pos_v.at[r]], sem_i).wait()

        def permg(g, carry):
            @pl.when(g > 0)
            def _():
                drain_group(g - 1)
            for rr in range(KGRP):
                r = g * KGRP + rr
                dcl = []
                for jj in range(SCAT // LANES):
                    d = digit_of(
                        loc_k[pl.ds(r * SCAT + jj * LANES, LANES)], shift)
                    cnt, last = plsc.scan_count(d)
                    dcl.append((d, cnt, last))
                for jj, (d, cnt, last) in enumerate(dcl):
                    pos = plsc.load_gather(offs_v, [d]) + cnt - 1
                    plsc.addupdate_scatter(offs_v, [d], cnt, mask=last)
                    pos_v[r, pl.ds(jj * LANES, LANES)] = pos
            for rr in range(KGRP):
                r = g * KGRP + rr
                pltpu.async_copy(loc_k.at[pl.ds(r * SCAT, SCAT)],
                                 dst_k.at[pos_v.at[r]], sem_k)
                pltpu.async_copy(loc_i.at[pl.ds(r * SCAT, SCAT)],
                                 dst_i.at[pos_v.at[r]], sem_i)
            return carry

        for h in range(NHALF):
            hbase = base + h * HALF
            pltpu.sync_copy(src_k.at[pl.ds(hbase, HALF)], loc_k)
            if first:
                def fill(j, carry):
                    loc_i[pl.ds(j * LANES, LANES)] = (
                        hbase + j * LANES + lax.iota(jnp.int32, LANES))
                    return carry
                lax.fori_loop(0, HVREG, fill, 0)
            else:
                pltpu.sync_copy(src_i.at[pl.ds(hbase, HALF)], loc_i)
            lax.fori_loop(0, HROWS // KGRP, permg, 0)
            drain_group(HROWS // KGRP - 1)
        plsc.subcore_barrier()

    run_pass(0, phi_hbm, None, sp_k0, sp_i0, True)
    run_pass(BITS, sp_k0, sp_i0, sp_k1, sp_i1, False)
    run_pass(2 * BITS, sp_k1, sp_i1, sp_k0, sp_i0, False)

    pltpu.sync_copy(sp_k0.at[pl.ds(base, CHUNK)],
                    out_phi.at[pl.ds(base, CHUNK)])
    pltpu.sync_copy(sp_i0.at[pl.ds(base, CHUNK)],
                    out_idx.at[pl.ds(base, CHUNK)])


@functools.partial(
    pl.kernel,
    out_type=jax.ShapeDtypeStruct((N, D), jnp.float32),
    mesh=_gather_mesh,
    compiler_params=pltpu.CompilerParams(
        needs_layout_passes=False, use_tc_tiling_on_sc=False),
    scratch_types=[
        pltpu.VMEM((NG, GC), jnp.int32),
        pltpu.VMEM((GC, D), jnp.float32),
        pltpu.VMEM((GC, D), jnp.float32),
        pltpu.VMEM((GC, D), jnp.float32),
        pltpu.VMEM((GC, D), jnp.float32),
        pltpu.SemaphoreType.DMA,
        pltpu.SemaphoreType.DMA,
        pltpu.SemaphoreType.DMA,
        pltpu.SemaphoreType.DMA,
    ],
)
def _gather_kernel(emb_hbm, idx_hbm, out_hbm, idx_v,
                   rows0, rows1, rows2, rows3, sem0, sem1, sem2, sem3):
    wid = lax.axis_index("s") * 2 + lax.axis_index("c")
    pltpu.sync_copy(idx_hbm.at[pl.ds(wid * NG, NG)], idx_v)
    base = wid * RPW
    rows = (rows0, rows1, rows2, rows3)
    sems = (sem0, sem1, sem2, sem3)
    DEPTH = 4

    for b in range(DEPTH - 1):
        pltpu.async_copy(emb_hbm.at[idx_v.at[b]], rows[b], sems[b])

    def g(gi, carry):
        r0 = DEPTH * gi
        for b in range(DEPTH):
            r = r0 + b
            nxt = r + DEPTH - 1

            @pl.when(nxt < NG)
            def _():
                pltpu.async_copy(emb_hbm.at[idx_v.at[nxt]],
                                 rows[(b + DEPTH - 1) % DEPTH],
                                 sems[(b + DEPTH - 1) % DEPTH])
            pltpu.make_async_copy(emb_hbm.at[idx_v.at[r]], rows[b],
                                  sems[b]).wait()
            pltpu.sync_copy(rows[b], out_hbm.at[pl.ds(base + r * GC, GC)])
        return carry
    lax.fori_loop(0, NG // DEPTH, g, 0)


def kernel(key_phi, key_embed):
    phi_bits = lax.bitcast_convert_type(key_phi.reshape(N), jnp.int32)
    emb = key_embed.reshape(N, D)
    sorted_bits, sort_idx = _sort_kernel(phi_bits)
    sorted_phi = lax.bitcast_convert_type(sorted_bits, jnp.float32)
    sorted_emb = _gather_kernel(emb, sort_idx.reshape(N // GC, GC))
    return sorted_phi.reshape(1, N), sorted_emb.reshape(1, N, D)


# speedup vs baseline: 1.3199x; 1.0007x over previous
import functools

import jax
import jax.numpy as jnp
from jax import lax
from jax.experimental import pallas as pl
from jax.experimental.pallas import tpu as pltpu
from jax.experimental.pallas import tpu_sc as plsc

N = 262144
D = 64
LANES = 16

NS = 16
CHUNK = N // NS
NHALF = 2
HALF = CHUNK // NHALF
HVREG = HALF // LANES
BITS = 10
BINS = 1 << BITS
SCAT = 128
HROWS = HALF // SCAT
KGRP = 8

NW = 32
RPW = N // NW
GC = 128
NG = RPW // GC

_sort_mesh = plsc.VectorSubcoreMesh(
    core_axis_name="c", subcore_axis_name="s", num_cores=1)
_gather_mesh = plsc.VectorSubcoreMesh(
    core_axis_name="c", subcore_axis_name="s")


@functools.partial(
    pl.kernel,
    out_type=(
        jax.ShapeDtypeStruct((N,), jnp.int32),
        jax.ShapeDtypeStruct((N,), jnp.int32),
    ),
    mesh=_sort_mesh,
    compiler_params=pltpu.CompilerParams(needs_layout_passes=False),
    scratch_types=[
        pltpu.VMEM((HALF,), jnp.int32),
        pltpu.VMEM((HALF,), jnp.int32),
        pltpu.VMEM((BINS,), jnp.int32),
        pltpu.VMEM((NS, BINS), jnp.int32),
        pltpu.VMEM((BINS,), jnp.int32),
        pltpu.VMEM((HROWS, SCAT), jnp.int32),
        pltpu.SemaphoreType.DMA,
        pltpu.SemaphoreType.DMA,
        pltpu.VMEM_SHARED((N,), jnp.int32),
        pltpu.VMEM_SHARED((N,), jnp.int32),
        pltpu.VMEM_SHARED((N,), jnp.int32),
        pltpu.VMEM_SHARED((N,), jnp.int32),
        pltpu.VMEM_SHARED((NS, BINS), jnp.int32),
    ],
)
def _sort_kernel(phi_hbm, out_phi, out_idx,
                 loc_k, loc_i, hist_v, grid_v, offs_v, pos_v, sem_k, sem_i,
                 sp_k0, sp_i0, sp_k1, sp_i1, sp_hist):
    t = lax.axis_index("s")
    base = t * CHUNK
    zeros16 = lax.full((LANES,), 0, jnp.int32)

    def digit_of(u, shift):
        return lax.shift_right_logical(u, lax.full((LANES,), shift, jnp.int32)) \
            & (BINS - 1)

    def run_pass(shift, src_k, src_i, dst_k, dst_i, first):
        def zeroh(j, carry):
            hist_v[pl.ds(j * LANES, LANES)] = zeros16
            return carry
        lax.fori_loop(0, BINS // LANES, zeroh, 0)

        def histg(g, carry):
            dcl = []
            for jj in range(SCAT // LANES):
                d = digit_of(loc_k[pl.ds(g * SCAT + jj * LANES, LANES)],
                             shift)
                cnt, last = plsc.scan_count(d)
                dcl.append((d, cnt, last))
            for d, cnt, last in dcl:
                plsc.addupdate_scatter(hist_v, [d], cnt, mask=last)
            return carry

        for h in range(NHALF):
            pltpu.sync_copy(src_k.at[pl.ds(base + h * HALF, HALF)], loc_k)
            lax.fori_loop(0, HROWS, histg, 0)

        pltpu.sync_copy(hist_v, sp_hist.at[t])
        plsc.subcore_barrier()
        pltpu.sync_copy(sp_hist, grid_v)

        def offj(j, carry):
            tot = zeros16
            rp = zeros16
            for tt in range(NS):
                row = grid_v[tt, pl.ds(j * LANES, LANES)]
                tot = tot + row
                w = jnp.where(tt < t, jnp.int32(1), jnp.int32(0))
                rp = rp + row * w
            excl = plsc.cumsum(tot) - tot
            offs_v[pl.ds(j * LANES, LANES)] = excl + rp + carry
            return carry + jnp.sum(tot)
        lax.fori_loop(0, BINS // LANES, offj, jnp.int32(0))

        def drain_group(g0):
            for rr in range(KGRP):
                r = g0 * KGRP + rr
                pltpu.make_async_copy(loc_k.at[pl.ds(r * SCAT, SCAT)],
                                      dst_k.at[pos_v.at[r]], sem_k).wait()
                pltpu.make_async_copy(loc_i.at[pl.ds(r * SCAT, SCAT)],
                                      dst_i.at[pos_v.at[r]], sem_i).wait()

        def permg(g, carry):
            @pl.when(g > 0)
            def _():
                drain_group(g - 1)
            for rr in range(KGRP):
                r = g * KGRP + rr
                dcl = []
                for jj in range(SCAT // LANES):
                    d = digit_of(
                        loc_k[pl.ds(r * SCAT + jj * LANES, LANES)], shift)
                    cnt, last = plsc.scan_count(d)
                    dcl.append((d, cnt, last))
                for jj, (d, cnt, last) in enumerate(dcl):
                    pos = plsc.load_gather(offs_v, [d]) + cnt - 1
                    plsc.addupdate_scatter(offs_v, [d], cnt, mask=last)
                    pos_v[r, pl.ds(jj * LANES, LANES)] = pos
            for rr in range(KGRP):
                r = g * KGRP + rr
                pltpu.async_copy(loc_k.at[pl.ds(r * SCAT, SCAT)],
                                 dst_k.at[pos_v.at[r]], sem_k)
                pltpu.async_copy(loc_i.at[pl.ds(r * SCAT, SCAT)],
                                 dst_i.at[pos_v.at[r]], sem_i)
            return carry

        for h in range(NHALF):
            hbase = base + h * HALF
            pltpu.sync_copy(src_k.at[pl.ds(hbase, HALF)], loc_k)
            if first:
                def fill(j, carry):
                    loc_i[pl.ds(j * LANES, LANES)] = (
                        hbase + j * LANES + lax.iota(jnp.int32, LANES))
                    return carry
                lax.fori_loop(0, HVREG, fill, 0)
            else:
                pltpu.sync_copy(src_i.at[pl.ds(hbase, HALF)], loc_i)
            lax.fori_loop(0, HROWS // KGRP, permg, 0)
            drain_group(HROWS // KGRP - 1)
        plsc.subcore_barrier()

    run_pass(0, phi_hbm, None, sp_k0, sp_i0, True)
    run_pass(BITS, sp_k0, sp_i0, sp_k1, sp_i1, False)
    run_pass(2 * BITS, sp_k1, sp_i1, sp_k0, sp_i0, False)

    pltpu.sync_copy(sp_k0.at[pl.ds(base, CHUNK)],
                    out_phi.at[pl.ds(base, CHUNK)])
    pltpu.sync_copy(sp_i0.at[pl.ds(base, CHUNK)],
                    out_idx.at[pl.ds(base, CHUNK)])


@functools.partial(
    pl.kernel,
    out_type=jax.ShapeDtypeStruct((N, D), jnp.float32),
    mesh=_gather_mesh,
    compiler_params=pltpu.CompilerParams(
        needs_layout_passes=False, use_tc_tiling_on_sc=False),
    scratch_types=[
        pltpu.VMEM((NG, GC), jnp.int32),
        pltpu.VMEM((GC, D), jnp.float32),
        pltpu.VMEM((GC, D), jnp.float32),
        pltpu.VMEM((GC, D), jnp.float32),
        pltpu.VMEM((GC, D), jnp.float32),
        pltpu.SemaphoreType.DMA,
        pltpu.SemaphoreType.DMA,
        pltpu.SemaphoreType.DMA,
        pltpu.SemaphoreType.DMA,
        pltpu.SemaphoreType.DMA,
        pltpu.SemaphoreType.DMA,
        pltpu.SemaphoreType.DMA,
        pltpu.SemaphoreType.DMA,
    ],
)
def _gather_kernel(emb_hbm, idx_hbm, out_hbm, idx_v,
                   rows0, rows1, rows2, rows3, sem0, sem1, sem2, sem3,
                   wsem0, wsem1, wsem2, wsem3):
    wid = lax.axis_index("s") * 2 + lax.axis_index("c")
    pltpu.sync_copy(idx_hbm.at[pl.ds(wid * NG, NG)], idx_v)
    base = wid * RPW
    rows = (rows0, rows1, rows2, rows3)
    sems = (sem0, sem1, sem2, sem3)
    wsems = (wsem0, wsem1, wsem2, wsem3)
    DEPTH = 4

    def wr_copy(r, b):
        return pltpu.make_async_copy(
            rows[b], out_hbm.at[pl.ds(base + r * GC, GC)], wsems[b])

    for b in range(DEPTH - 1):
        pltpu.async_copy(emb_hbm.at[idx_v.at[b]], rows[b], sems[b])

    def g(gi, carry):
        r0 = DEPTH * gi
        for b in range(DEPTH):
            r = r0 + b
            nxt = r + DEPTH - 1
            nb = (b + DEPTH - 1) % DEPTH

            @pl.when(nxt < NG)
            def _():
                @pl.when(nxt >= DEPTH)
                def _():
                    wr_copy(nxt - DEPTH, nb).wait()
                pltpu.async_copy(emb_hbm.at[idx_v.at[nxt]], rows[nb],
                                 sems[nb])
            pltpu.make_async_copy(emb_hbm.at[idx_v.at[r]], rows[b],
                                  sems[b]).wait()
            wr_copy(r, b).start()
        return carry
    lax.fori_loop(0, NG // DEPTH, g, 0)
    for k in range(DEPTH):
        wr_copy(NG - DEPTH + k, (NG - DEPTH + k) % DEPTH).wait()


def kernel(key_phi, key_embed):
    phi_bits = lax.bitcast_convert_type(key_phi.reshape(N), jnp.int32)
    emb = key_embed.reshape(N, D)
    sorted_bits, sort_idx = _sort_kernel(phi_bits)
    sorted_phi = lax.bitcast_convert_type(sorted_bits, jnp.float32)
    sorted_emb = _gather_kernel(emb, sort_idx.reshape(N // GC, GC))
    return sorted_phi.reshape(1, N), sorted_emb.reshape(1, N, D)
